# Initial kernel scaffold; baseline (speedup 1.0000x reference)
import jax
import jax.numpy as jnp
from jax.experimental import pallas as pl


def kernel(input, suplabel, seed_h, seed_w, seed_level):
    raise NotImplementedError("write your pallas kernel here")


# speedup vs baseline: 65.4464x; 65.4464x over previous
import jax
import jax.numpy as jnp
from jax import lax
from jax.experimental import pallas as pl
from jax.experimental.pallas import tpu as pltpu
from jax.experimental.pallas import tpu_sc as plsc

BB = 8
CC = 3
KK = 1024
PP = 512 * 512
GROUP = 4
PPT = PP // GROUP
CH = 8192
NCHUNK = PPT // CH
L = 16


def _full_body(inp, lab, out, lbl_v, r_v, g_v, b_v, acc_r, acc_g, acc_b,
               acc_n, shared, tmp, outv):
    c = lax.axis_index("c")
    s = lax.axis_index("s")
    batch = c * (16 // GROUP) + s // GROUP
    q = s % GROUP
    base = q * PPT

    zeros = jnp.zeros((L,), jnp.float32)

    def zbody(j, _):
        o = j * L
        acc_r[pl.ds(o, L)] = zeros
        acc_g[pl.ds(o, L)] = zeros
        acc_b[pl.ds(o, L)] = zeros
        acc_n[pl.ds(o, L)] = zeros
        return 0

    lax.fori_loop(0, KK // L, zbody, 0)

    ones = jnp.full((L,), 1.0, jnp.float32)
    for chunk in range(NCHUNK):
        start = base + chunk * CH
        pltpu.sync_copy(lab.at[pl.ds(batch * PP + start, CH)], lbl_v)
        pltpu.sync_copy(inp.at[pl.ds((batch * CC + 0) * PP + start, CH)], r_v)
        pltpu.sync_copy(inp.at[pl.ds((batch * CC + 1) * PP + start, CH)], g_v)
        pltpu.sync_copy(inp.at[pl.ds((batch * CC + 2) * PP + start, CH)], b_v)

        def ibody(i, _):
            o = i * L
            idx = lbl_v[pl.ds(o, L)]
            plsc.addupdate_scatter(acc_r, [idx], r_v[pl.ds(o, L)])
            plsc.addupdate_scatter(acc_g, [idx], g_v[pl.ds(o, L)])
            plsc.addupdate_scatter(acc_b, [idx], b_v[pl.ds(o, L)])
            plsc.addupdate_scatter(acc_n, [idx], ones)
            return 0

        lax.fori_loop(0, CH // L, ibody, 0)

    pltpu.sync_copy(acc_r, shared.at[pl.ds(s * 4 * KK + 0 * KK, KK)])
    pltpu.sync_copy(acc_g, shared.at[pl.ds(s * 4 * KK + 1 * KK, KK)])
    pltpu.sync_copy(acc_b, shared.at[pl.ds(s * 4 * KK + 2 * KK, KK)])
    pltpu.sync_copy(acc_n, shared.at[pl.ds(s * 4 * KK + 3 * KK, KK)])
    plsc.subcore_barrier()

    @pl.when(q == 0)
    def _leader():
        for n in range(1, GROUP):
            pltpu.sync_copy(shared.at[pl.ds((s + n) * 4 * KK, 4 * KK)], tmp)

            def abody(j, _):
                o = j * L
                acc_r[pl.ds(o, L)] += tmp[pl.ds(0 * KK + o, L)]
                acc_g[pl.ds(o, L)] += tmp[pl.ds(1 * KK + o, L)]
                acc_b[pl.ds(o, L)] += tmp[pl.ds(2 * KK + o, L)]
                acc_n[pl.ds(o, L)] += tmp[pl.ds(3 * KK + o, L)]
                return 0

            lax.fori_loop(0, KK // L, abody, 0)

        def mbody(j, _):
            o = j * L
            d = jnp.maximum(acc_n[pl.ds(o, L)], 1.0)
            outv[pl.ds(0 * KK + o, L)] = acc_r[pl.ds(o, L)] / d
            outv[pl.ds(1 * KK + o, L)] = acc_g[pl.ds(o, L)] / d
            outv[pl.ds(2 * KK + o, L)] = acc_b[pl.ds(o, L)] / d
            return 0

        lax.fori_loop(0, KK // L, mbody, 0)
        pltpu.sync_copy(outv, out.at[pl.ds(batch * CC * KK, CC * KK)])


@jax.jit
def _superpixel_color(inp, lab):
    mesh = plsc.VectorSubcoreMesh(core_axis_name="c", subcore_axis_name="s")
    f = pl.kernel(
        _full_body,
        out_type=jax.ShapeDtypeStruct((BB * CC * KK,), jnp.float32),
        mesh=mesh,
        compiler_params=pltpu.CompilerParams(needs_layout_passes=False),
        scratch_types=[
            pltpu.VMEM((CH,), jnp.int32),
            pltpu.VMEM((CH,), jnp.float32),
            pltpu.VMEM((CH,), jnp.float32),
            pltpu.VMEM((CH,), jnp.float32),
            pltpu.VMEM((KK,), jnp.float32),
            pltpu.VMEM((KK,), jnp.float32),
            pltpu.VMEM((KK,), jnp.float32),
            pltpu.VMEM((KK,), jnp.float32),
            pltpu.VMEM_SHARED((16 * 4 * KK,), jnp.float32),
            pltpu.VMEM((4 * KK,), jnp.float32),
            pltpu.VMEM((CC * KK,), jnp.float32),
        ],
    )
    return f(inp, lab)


def kernel(input, suplabel, seed_h, seed_w, seed_level):
    b, ch, h, w = input.shape
    inp = input.reshape(b * ch * h * w)
    lab = suplabel.reshape(b * h * w).astype(jnp.int32)
    out = _superpixel_color(inp, lab)
    return out.reshape(b, ch, KK).transpose(0, 2, 1)


# speedup vs baseline: 80.4923x; 1.2299x over previous
import jax
import jax.numpy as jnp
from jax import lax
from jax.experimental import pallas as pl
from jax.experimental.pallas import tpu as pltpu
from jax.experimental.pallas import tpu_sc as plsc

BB = 8
CC = 3
KK = 1024
PP = 512 * 512
GROUP = 4
PPT = PP // GROUP
CH = 8192
NCHUNK = PPT // CH
L = 16
UNROLL = 8


def _full_body(inp, lab, out, lbl0, r0, g0, b0, lbl1, r1, g1, b1,
               acc_r, acc_g, acc_b, acc_n, shared, tmp, outv, sem0, sem1):
    c = lax.axis_index("c")
    s = lax.axis_index("s")
    batch = c * (16 // GROUP) + s // GROUP
    q = s % GROUP
    base = q * PPT

    bufs = ((lbl0, r0, g0, b0), (lbl1, r1, g1, b1))
    sems = (sem0, sem1)

    zeros = jnp.zeros((L,), jnp.float32)

    def zbody(j, _):
        o = j * L
        acc_r[pl.ds(o, L)] = zeros
        acc_g[pl.ds(o, L)] = zeros
        acc_b[pl.ds(o, L)] = zeros
        acc_n[pl.ds(o, L)] = zeros
        return 0

    lax.fori_loop(0, KK // L, zbody, 0)

    def issue(chunk, bi):
        start = base + chunk * CH
        lb, rv, gv, bv = bufs[bi]
        sem = sems[bi]
        return (
            pltpu.async_copy(lab.at[pl.ds(batch * PP + start, CH)], lb, sem),
            pltpu.async_copy(
                inp.at[pl.ds((batch * CC + 0) * PP + start, CH)], rv, sem),
            pltpu.async_copy(
                inp.at[pl.ds((batch * CC + 1) * PP + start, CH)], gv, sem),
            pltpu.async_copy(
                inp.at[pl.ds((batch * CC + 2) * PP + start, CH)], bv, sem),
        )

    ones = jnp.full((L,), 1.0, jnp.float32)
    pending = issue(0, 0)
    for chunk in range(NCHUNK):
        bi = chunk % 2
        cur = pending
        if chunk + 1 < NCHUNK:
            pending = issue(chunk + 1, 1 - bi)
        for h in cur:
            h.wait()
        lb, rv, gv, bv = bufs[bi]

        def ibody(i, _):
            o0 = i * (L * UNROLL)
            for u in range(UNROLL):
                o = o0 + u * L
                idx = lb[pl.ds(o, L)]
                plsc.addupdate_scatter(acc_r, [idx], rv[pl.ds(o, L)])
                plsc.addupdate_scatter(acc_g, [idx], gv[pl.ds(o, L)])
                plsc.addupdate_scatter(acc_b, [idx], bv[pl.ds(o, L)])
                plsc.addupdate_scatter(acc_n, [idx], ones)
            return 0

        lax.fori_loop(0, CH // (L * UNROLL), ibody, 0)

    pltpu.sync_copy(acc_r, shared.at[pl.ds(s * 4 * KK + 0 * KK, KK)])
    pltpu.sync_copy(acc_g, shared.at[pl.ds(s * 4 * KK + 1 * KK, KK)])
    pltpu.sync_copy(acc_b, shared.at[pl.ds(s * 4 * KK + 2 * KK, KK)])
    pltpu.sync_copy(acc_n, shared.at[pl.ds(s * 4 * KK + 3 * KK, KK)])
    plsc.subcore_barrier()

    @pl.when(q == 0)
    def _leader():
        for n in range(1, GROUP):
            pltpu.sync_copy(shared.at[pl.ds((s + n) * 4 * KK, 4 * KK)], tmp)

            def abody(j, _):
                o = j * L
                acc_r[pl.ds(o, L)] += tmp[pl.ds(0 * KK + o, L)]
                acc_g[pl.ds(o, L)] += tmp[pl.ds(1 * KK + o, L)]
                acc_b[pl.ds(o, L)] += tmp[pl.ds(2 * KK + o, L)]
                acc_n[pl.ds(o, L)] += tmp[pl.ds(3 * KK + o, L)]
                return 0

            lax.fori_loop(0, KK // L, abody, 0)

        def mbody(j, _):
            o = j * L
            d = jnp.maximum(acc_n[pl.ds(o, L)], 1.0)
            outv[pl.ds(0 * KK + o, L)] = acc_r[pl.ds(o, L)] / d
            outv[pl.ds(1 * KK + o, L)] = acc_g[pl.ds(o, L)] / d
            outv[pl.ds(2 * KK + o, L)] = acc_b[pl.ds(o, L)] / d
            return 0

        lax.fori_loop(0, KK // L, mbody, 0)
        pltpu.sync_copy(outv, out.at[pl.ds(batch * CC * KK, CC * KK)])


@jax.jit
def _superpixel_color(inp, lab):
    mesh = plsc.VectorSubcoreMesh(core_axis_name="c", subcore_axis_name="s")
    f = pl.kernel(
        _full_body,
        out_type=jax.ShapeDtypeStruct((BB * CC * KK,), jnp.float32),
        mesh=mesh,
        compiler_params=pltpu.CompilerParams(needs_layout_passes=False),
        scratch_types=[
            pltpu.VMEM((CH,), jnp.int32),
            pltpu.VMEM((CH,), jnp.float32),
            pltpu.VMEM((CH,), jnp.float32),
            pltpu.VMEM((CH,), jnp.float32),
            pltpu.VMEM((CH,), jnp.int32),
            pltpu.VMEM((CH,), jnp.float32),
            pltpu.VMEM((CH,), jnp.float32),
            pltpu.VMEM((CH,), jnp.float32),
            pltpu.VMEM((KK,), jnp.float32),
            pltpu.VMEM((KK,), jnp.float32),
            pltpu.VMEM((KK,), jnp.float32),
            pltpu.VMEM((KK,), jnp.float32),
            pltpu.VMEM_SHARED((16 * 4 * KK,), jnp.float32),
            pltpu.VMEM((4 * KK,), jnp.float32),
            pltpu.VMEM((CC * KK,), jnp.float32),
            pltpu.SemaphoreType.DMA,
            pltpu.SemaphoreType.DMA,
        ],
    )
    return f(inp, lab)


def kernel(input, suplabel, seed_h, seed_w, seed_level):
    b, ch, h, w = input.shape
    inp = input.reshape(b * ch * h * w)
    lab = suplabel.reshape(b * h * w).astype(jnp.int32)
    out = _superpixel_color(inp, lab)
    return out.reshape(b, ch, KK).transpose(0, 2, 1)


# speedup vs baseline: 100.4840x; 1.2484x over previous
import jax
import jax.numpy as jnp
from jax import lax
from jax.experimental import pallas as pl
from jax.experimental.pallas import tpu as pltpu
from jax.experimental.pallas import tpu_sc as plsc

BB = 8
CC = 3
KK = 1024
HH = 512
WW = 512
GROUP = 4
RPT = HH // GROUP
RCH = 16
NCHUNK = RPT // RCH
L = 16


def _full_body(inp, lab, out, lbl0, r0, g0, b0, lbl1, r1, g1, b1,
               acc_r, acc_g, acc_b, acc_n, shared, tmp, outv, sem0, sem1):
    c = lax.axis_index("c")
    s = lax.axis_index("s")
    batch = c * (16 // GROUP) + s // GROUP
    q = s % GROUP
    base_row = q * RPT

    bufs = ((lbl0, r0, g0, b0), (lbl1, r1, g1, b1))
    sems = (sem0, sem1)

    zeros = jnp.zeros((L,), jnp.float32)

    def zbody(j, _):
        o = j * L
        acc_r[pl.ds(o, L)] = zeros
        acc_g[pl.ds(o, L)] = zeros
        acc_b[pl.ds(o, L)] = zeros
        acc_n[pl.ds(o, L)] = zeros
        return 0

    lax.fori_loop(0, KK // L, zbody, 0)

    def issue(chunk, bi):
        row = base_row + chunk * RCH
        lb, rv, gv, bv = bufs[bi]
        sem = sems[bi]
        return (
            pltpu.async_copy(lab.at[batch, pl.ds(row, RCH), :], lb, sem),
            pltpu.async_copy(inp.at[batch, 0, pl.ds(row, RCH), :], rv, sem),
            pltpu.async_copy(inp.at[batch, 1, pl.ds(row, RCH), :], gv, sem),
            pltpu.async_copy(inp.at[batch, 2, pl.ds(row, RCH), :], bv, sem),
        )

    ones = jnp.full((L,), 1.0, jnp.float32)
    pending = issue(0, 0)
    for chunk in range(NCHUNK):
        bi = chunk % 2
        cur = pending
        if chunk + 1 < NCHUNK:
            pending = issue(chunk + 1, 1 - bi)
        for h in cur:
            h.wait()
        lb, rv, gv, bv = bufs[bi]

        def ibody(j, _):
            o = j * L
            for row in range(RCH):
                idx = lb[row, pl.ds(o, L)]
                plsc.addupdate_scatter(acc_r, [idx], rv[row, pl.ds(o, L)])
                plsc.addupdate_scatter(acc_g, [idx], gv[row, pl.ds(o, L)])
                plsc.addupdate_scatter(acc_b, [idx], bv[row, pl.ds(o, L)])
                plsc.addupdate_scatter(acc_n, [idx], ones)
            return 0

        lax.fori_loop(0, WW // L, ibody, 0)

    pltpu.sync_copy(acc_r, shared.at[pl.ds(s * 4 * KK + 0 * KK, KK)])
    pltpu.sync_copy(acc_g, shared.at[pl.ds(s * 4 * KK + 1 * KK, KK)])
    pltpu.sync_copy(acc_b, shared.at[pl.ds(s * 4 * KK + 2 * KK, KK)])
    pltpu.sync_copy(acc_n, shared.at[pl.ds(s * 4 * KK + 3 * KK, KK)])
    plsc.subcore_barrier()

    @pl.when(q == 0)
    def _leader():
        for n in range(1, GROUP):
            pltpu.sync_copy(shared.at[pl.ds((s + n) * 4 * KK, 4 * KK)], tmp)

            def abody(j, _):
                o = j * L
                acc_r[pl.ds(o, L)] += tmp[pl.ds(0 * KK + o, L)]
                acc_g[pl.ds(o, L)] += tmp[pl.ds(1 * KK + o, L)]
                acc_b[pl.ds(o, L)] += tmp[pl.ds(2 * KK + o, L)]
                acc_n[pl.ds(o, L)] += tmp[pl.ds(3 * KK + o, L)]
                return 0

            lax.fori_loop(0, KK // L, abody, 0)

        iota3 = lax.iota(jnp.int32, L) * CC

        def mbody(j, _):
            o = j * L
            d = jnp.maximum(acc_n[pl.ds(o, L)], 1.0)
            idx = iota3 + (CC * o)
            plsc.store_scatter(outv, [idx], acc_r[pl.ds(o, L)] / d)
            plsc.store_scatter(outv, [idx + 1], acc_g[pl.ds(o, L)] / d)
            plsc.store_scatter(outv, [idx + 2], acc_b[pl.ds(o, L)] / d)
            return 0

        lax.fori_loop(0, KK // L, mbody, 0)
        pltpu.sync_copy(outv, out.at[pl.ds(batch * CC * KK, CC * KK)])


@jax.jit
def _superpixel_color(inp, lab):
    mesh = plsc.VectorSubcoreMesh(core_axis_name="c", subcore_axis_name="s")
    f = pl.kernel(
        _full_body,
        out_type=jax.ShapeDtypeStruct((BB * KK * CC,), jnp.float32),
        mesh=mesh,
        compiler_params=pltpu.CompilerParams(needs_layout_passes=False),
        scratch_types=[
            pltpu.VMEM((RCH, WW), jnp.int32),
            pltpu.VMEM((RCH, WW), jnp.float32),
            pltpu.VMEM((RCH, WW), jnp.float32),
            pltpu.VMEM((RCH, WW), jnp.float32),
            pltpu.VMEM((RCH, WW), jnp.int32),
            pltpu.VMEM((RCH, WW), jnp.float32),
            pltpu.VMEM((RCH, WW), jnp.float32),
            pltpu.VMEM((RCH, WW), jnp.float32),
            pltpu.VMEM((KK,), jnp.float32),
            pltpu.VMEM((KK,), jnp.float32),
            pltpu.VMEM((KK,), jnp.float32),
            pltpu.VMEM((KK,), jnp.float32),
            pltpu.VMEM_SHARED((16 * 4 * KK,), jnp.float32),
            pltpu.VMEM((4 * KK,), jnp.float32),
            pltpu.VMEM((KK * CC,), jnp.float32),
            pltpu.SemaphoreType.DMA,
            pltpu.SemaphoreType.DMA,
        ],
    )
    return f(inp, lab)


def kernel(input, suplabel, seed_h, seed_w, seed_level):
    b, ch, h, w = input.shape
    lab = suplabel.astype(jnp.int32)
    out = _superpixel_color(input, lab)
    return out.reshape(b, KK, ch)


# speedup vs baseline: 142.1463x; 1.4146x over previous
import jax
import jax.numpy as jnp
from jax import lax
from jax.experimental import pallas as pl
from jax.experimental.pallas import tpu as pltpu
from jax.experimental.pallas import tpu_sc as plsc

BB = 8
CC = 3
KK = 1024
HH = 512
WW = 512
GROUP = 4
RPT = HH // GROUP
RCH = 16
NCHUNK = RPT // RCH
L = 16


def _full_body(inp, lab, out, lbl0, r0, g0, b0, lbl1, r1, g1, b1,
               acc_r, acc_g, acc_b, acc_n, shared, tmp, outv, sem0, sem1):
    c = lax.axis_index("c")
    s = lax.axis_index("s")
    batch = c * (16 // GROUP) + s // GROUP
    q = s % GROUP
    base_row = q * RPT

    bufs = ((lbl0, r0, g0, b0), (lbl1, r1, g1, b1))
    sems = (sem0, sem1)

    zeros = jnp.zeros((L,), jnp.float32)

    def zbody(j, _):
        o = j * L
        acc_r[pl.ds(o, L)] = zeros
        acc_g[pl.ds(o, L)] = zeros
        acc_b[pl.ds(o, L)] = zeros
        acc_n[pl.ds(o, L)] = zeros
        return 0

    lax.fori_loop(0, KK // L, zbody, 0)

    def issue(chunk, bi):
        row = base_row + chunk * RCH
        lb, rv, gv, bv = bufs[bi]
        sem = sems[bi]
        return (
            pltpu.async_copy(lab.at[batch, pl.ds(row, RCH), :], lb, sem),
            pltpu.async_copy(inp.at[batch, 0, pl.ds(row, RCH), :], rv, sem),
            pltpu.async_copy(inp.at[batch, 1, pl.ds(row, RCH), :], gv, sem),
            pltpu.async_copy(inp.at[batch, 2, pl.ds(row, RCH), :], bv, sem),
        )

    ones = jnp.full((L,), 1.0, jnp.float32)
    pending = issue(0, 0)
    for chunk in range(NCHUNK):
        bi = chunk % 2
        cur = pending
        if chunk + 1 < NCHUNK:
            pending = issue(chunk + 1, 1 - bi)
        for h in cur:
            h.wait()
        lb, rv, gv, bv = bufs[bi]

        def ibody(j, _):
            o = j * L
            for rb in range(0, RCH, 4):
                vals = []
                for row in range(rb, rb + 4):
                    vals.append((lb[row, pl.ds(o, L)],
                                 rv[row, pl.ds(o, L)],
                                 gv[row, pl.ds(o, L)],
                                 bv[row, pl.ds(o, L)]))
                for idx, rr, gg, bb in vals:
                    plsc.addupdate_scatter(acc_r, [idx], rr)
                    plsc.addupdate_scatter(acc_g, [idx], gg)
                    plsc.addupdate_scatter(acc_b, [idx], bb)
                    plsc.addupdate_scatter(acc_n, [idx], ones)
            return 0

        lax.fori_loop(0, WW // L, ibody, 0)

    pltpu.sync_copy(acc_r, shared.at[pl.ds(s * 4 * KK + 0 * KK, KK)])
    pltpu.sync_copy(acc_g, shared.at[pl.ds(s * 4 * KK + 1 * KK, KK)])
    pltpu.sync_copy(acc_b, shared.at[pl.ds(s * 4 * KK + 2 * KK, KK)])
    pltpu.sync_copy(acc_n, shared.at[pl.ds(s * 4 * KK + 3 * KK, KK)])
    plsc.subcore_barrier()

    @pl.when(q == 0)
    def _leader():
        for n in range(1, GROUP):
            pltpu.sync_copy(shared.at[pl.ds((s + n) * 4 * KK, 4 * KK)], tmp)

            def abody(j, _):
                o = j * L
                acc_r[pl.ds(o, L)] += tmp[pl.ds(0 * KK + o, L)]
                acc_g[pl.ds(o, L)] += tmp[pl.ds(1 * KK + o, L)]
                acc_b[pl.ds(o, L)] += tmp[pl.ds(2 * KK + o, L)]
                acc_n[pl.ds(o, L)] += tmp[pl.ds(3 * KK + o, L)]
                return 0

            lax.fori_loop(0, KK // L, abody, 0)

        iota3 = lax.iota(jnp.int32, L) * CC

        def mbody(j, _):
            o = j * L
            d = jnp.maximum(acc_n[pl.ds(o, L)], 1.0)
            idx = iota3 + (CC * o)
            plsc.store_scatter(outv, [idx], acc_r[pl.ds(o, L)] / d)
            plsc.store_scatter(outv, [idx + 1], acc_g[pl.ds(o, L)] / d)
            plsc.store_scatter(outv, [idx + 2], acc_b[pl.ds(o, L)] / d)
            return 0

        lax.fori_loop(0, KK // L, mbody, 0)
        pltpu.sync_copy(outv, out.at[pl.ds(batch * CC * KK, CC * KK)])


@jax.jit
def _superpixel_color(inp, lab):
    mesh = plsc.VectorSubcoreMesh(core_axis_name="c", subcore_axis_name="s")
    f = pl.kernel(
        _full_body,
        out_type=jax.ShapeDtypeStruct((BB * KK * CC,), jnp.float32),
        mesh=mesh,
        compiler_params=pltpu.CompilerParams(needs_layout_passes=False),
        scratch_types=[
            pltpu.VMEM((RCH, WW), jnp.int32),
            pltpu.VMEM((RCH, WW), jnp.float32),
            pltpu.VMEM((RCH, WW), jnp.float32),
            pltpu.VMEM((RCH, WW), jnp.float32),
            pltpu.VMEM((RCH, WW), jnp.int32),
            pltpu.VMEM((RCH, WW), jnp.float32),
            pltpu.VMEM((RCH, WW), jnp.float32),
            pltpu.VMEM((RCH, WW), jnp.float32),
            pltpu.VMEM((KK,), jnp.float32),
            pltpu.VMEM((KK,), jnp.float32),
            pltpu.VMEM((KK,), jnp.float32),
            pltpu.VMEM((KK,), jnp.float32),
            pltpu.VMEM_SHARED((16 * 4 * KK,), jnp.float32),
            pltpu.VMEM((4 * KK,), jnp.float32),
            pltpu.VMEM((KK * CC,), jnp.float32),
            pltpu.SemaphoreType.DMA,
            pltpu.SemaphoreType.DMA,
        ],
    )
    return f(inp, lab)


def kernel(input, suplabel, seed_h, seed_w, seed_level):
    b, ch, h, w = input.shape
    lab = suplabel.astype(jnp.int32)
    out = _superpixel_color(input, lab)
    return out.reshape(b, KK, ch)
